# baseline (speedup 1.0000x reference)
import functools

import jax
import jax.numpy as jnp
from jax import lax
from jax.experimental import pallas as pl
from jax.experimental.pallas import tpu as pltpu
from jax.experimental.pallas import tpu_sc as plsc

_NUM_WORKERS = 32
_LANES = 16
_NCHUNK = 4


def _round16(x):
    return (x + _LANES - 1) // _LANES * _LANES


@functools.lru_cache(maxsize=None)
def _build_sc_exp_map(n: int):
    cw = _round16(-(-(-(-n // _NUM_WORKERS)) // _NCHUNK))
    c_std = _NCHUNK * cw
    last_start = (_NUM_WORKERS - 1) * c_std
    c_last = n - last_start
    assert 0 < c_last <= c_std
    last_lens = []
    rem = c_last
    for _ in range(_NCHUNK):
        take = min(cw, rem)
        last_lens.append(take)
        rem -= take
    last_lens = tuple(last_lens)
    assert rem == 0

    mesh = plsc.VectorSubcoreMesh(core_axis_name="c", subcore_axis_name="s")

    @functools.partial(
        pl.kernel,
        out_type=jax.ShapeDtypeStruct((n,), jnp.float32),
        mesh=mesh,
        scratch_types=(
            [pltpu.VMEM((cw,), jnp.float32) for _ in range(_NCHUNK)]
            + [pltpu.VMEM((_LANES,), jnp.float32)]
            + [pltpu.SemaphoreType.DMA for _ in range(2 * _NCHUNK)]
        ),
    )
    def run(vals, scale, out, *scratch):
        bufs = scratch[:_NCHUNK]
        scale_v = scratch[_NCHUNK]
        sem_in = scratch[_NCHUNK + 1 : 2 * _NCHUNK + 1]
        sem_out = scratch[2 * _NCHUNK + 1 :]
        wid = lax.axis_index("c") * 16 + lax.axis_index("s")

        def pipeline(base, chunk_lens):
            def in_copy(k):
                w = chunk_lens[k]
                return pltpu.make_async_copy(
                    vals.at[pl.ds(base + k * cw, w)],
                    bufs[k].at[pl.ds(0, w)],
                    sem_in[k],
                )

            def out_copy(k):
                w = chunk_lens[k]
                return pltpu.make_async_copy(
                    bufs[k].at[pl.ds(0, w)],
                    out.at[pl.ds(base + k * cw, w)],
                    sem_out[k],
                )

            def compute(k, s):
                buf = bufs[k]
                w = _round16(chunk_lens[k])

                @plsc.parallel_loop(0, w, step=_LANES, unroll=8)
                def _(i):
                    o = pl.multiple_of(i, _LANES)
                    buf[pl.ds(o, _LANES)] = jnp.exp(buf[pl.ds(o, _LANES)] * s)

            live = [k for k in range(_NCHUNK) if chunk_lens[k] > 0]
            for k in live:
                in_copy(k).start()
            pltpu.sync_copy(scale, scale_v)
            s = scale_v[...]
            for k in live:
                in_copy(k).wait()
                compute(k, s)
                out_copy(k).start()
            for k in live:
                out_copy(k).wait()

        @pl.when(wid < _NUM_WORKERS - 1)
        def _():
            pipeline(wid * c_std, (cw,) * _NCHUNK)

        @pl.when(wid == _NUM_WORKERS - 1)
        def _():
            pipeline(last_start, last_lens)

    return run


def kernel(indices, values, rationality):
    del indices
    run = _build_sc_exp_map(values.shape[0])
    scale = jnp.full((_LANES,), -rationality, dtype=jnp.float32)
    return run(values, scale)


# speedup vs baseline: 1.0018x; 1.0018x over previous
import functools

import jax
import jax.numpy as jnp
from jax import lax
from jax.experimental import pallas as pl
from jax.experimental.pallas import tpu as pltpu
from jax.experimental.pallas import tpu_sc as plsc

_NUM_WORKERS = 32
_LANES = 16
_NCHUNK = 3


def _round16(x):
    return (x + _LANES - 1) // _LANES * _LANES


@functools.lru_cache(maxsize=None)
def _build_sc_exp_map(n: int):
    cw = _round16(-(-(-(-n // _NUM_WORKERS)) // _NCHUNK))
    c_std = _NCHUNK * cw
    last_start = (_NUM_WORKERS - 1) * c_std
    c_last = n - last_start
    assert 0 < c_last <= c_std
    last_lens = []
    rem = c_last
    for _ in range(_NCHUNK):
        take = min(cw, rem)
        last_lens.append(take)
        rem -= take
    last_lens = tuple(last_lens)
    assert rem == 0

    mesh = plsc.VectorSubcoreMesh(core_axis_name="c", subcore_axis_name="s")

    @functools.partial(
        pl.kernel,
        out_type=jax.ShapeDtypeStruct((n,), jnp.float32),
        mesh=mesh,
        scratch_types=(
            [pltpu.VMEM((cw,), jnp.float32) for _ in range(_NCHUNK)]
            + [pltpu.VMEM((_LANES,), jnp.float32)]
            + [pltpu.SemaphoreType.DMA for _ in range(2 * _NCHUNK)]
        ),
    )
    def run(vals, scale, out, *scratch):
        bufs = scratch[:_NCHUNK]
        scale_v = scratch[_NCHUNK]
        sem_in = scratch[_NCHUNK + 1 : 2 * _NCHUNK + 1]
        sem_out = scratch[2 * _NCHUNK + 1 :]
        wid = lax.axis_index("c") * 16 + lax.axis_index("s")

        def pipeline(base, chunk_lens):
            def in_copy(k):
                w = chunk_lens[k]
                return pltpu.make_async_copy(
                    vals.at[pl.ds(base + k * cw, w)],
                    bufs[k].at[pl.ds(0, w)],
                    sem_in[k],
                )

            def out_copy(k):
                w = chunk_lens[k]
                return pltpu.make_async_copy(
                    bufs[k].at[pl.ds(0, w)],
                    out.at[pl.ds(base + k * cw, w)],
                    sem_out[k],
                )

            def compute(k, s):
                buf = bufs[k]
                w = _round16(chunk_lens[k])

                @plsc.parallel_loop(0, w, step=_LANES, unroll=16)
                def _(i):
                    o = pl.multiple_of(i, _LANES)
                    buf[pl.ds(o, _LANES)] = jnp.exp(buf[pl.ds(o, _LANES)] * s)

            live = [k for k in range(_NCHUNK) if chunk_lens[k] > 0]
            for k in live:
                in_copy(k).start()
            pltpu.sync_copy(scale, scale_v)
            s = scale_v[...]
            for k in live:
                in_copy(k).wait()
                compute(k, s)
                out_copy(k).start()
            for k in live:
                out_copy(k).wait()

        @pl.when(wid < _NUM_WORKERS - 1)
        def _():
            pipeline(wid * c_std, (cw,) * _NCHUNK)

        @pl.when(wid == _NUM_WORKERS - 1)
        def _():
            pipeline(last_start, last_lens)

    return run


def kernel(indices, values, rationality):
    del indices
    run = _build_sc_exp_map(values.shape[0])
    scale = jnp.full((_LANES,), -rationality, dtype=jnp.float32)
    return run(values, scale)


# speedup vs baseline: 1.0276x; 1.0258x over previous
import functools

import jax
import jax.numpy as jnp
from jax import lax
from jax.experimental import pallas as pl
from jax.experimental.pallas import tpu as pltpu
from jax.experimental.pallas import tpu_sc as plsc

_NUM_WORKERS = 32
_LANES = 16
_NCHUNK = 3


def _round16(x):
    return (x + _LANES - 1) // _LANES * _LANES


@functools.lru_cache(maxsize=None)
def _build_sc_exp_map(n: int):
    cw = _round16(-(-(-(-n // _NUM_WORKERS)) // _NCHUNK))
    c_std = _NCHUNK * cw
    last_start = (_NUM_WORKERS - 1) * c_std
    c_last = n - last_start
    assert 0 < c_last <= c_std
    last_lens = []
    rem = c_last
    for _ in range(_NCHUNK):
        take = min(cw, rem)
        last_lens.append(take)
        rem -= take
    last_lens = tuple(last_lens)
    assert rem == 0

    mesh = plsc.VectorSubcoreMesh(core_axis_name="c", subcore_axis_name="s")

    @functools.partial(
        pl.kernel,
        out_type=jax.ShapeDtypeStruct((n,), jnp.float32),
        mesh=mesh,
        scratch_types=(
            [pltpu.VMEM((cw,), jnp.float32) for _ in range(_NCHUNK)]
            + [pltpu.VMEM((_LANES,), jnp.float32)]
            + [pltpu.SemaphoreType.DMA for _ in range(2 * _NCHUNK)]
        ),
    )
    def run(vals, scale, out, *scratch):
        bufs = scratch[:_NCHUNK]
        scale_v = scratch[_NCHUNK]
        sem_in = scratch[_NCHUNK + 1 : 2 * _NCHUNK + 1]
        sem_out = scratch[2 * _NCHUNK + 1 :]
        wid = lax.axis_index("c") * 16 + lax.axis_index("s")

        def pipeline(base, chunk_lens):
            def in_copy(k):
                w = chunk_lens[k]
                return pltpu.make_async_copy(
                    vals.at[pl.ds(base + k * cw, w)],
                    bufs[k].at[pl.ds(0, w)],
                    sem_in[k],
                )

            def out_copy(k):
                w = chunk_lens[k]
                return pltpu.make_async_copy(
                    bufs[k].at[pl.ds(0, w)],
                    out.at[pl.ds(base + k * cw, w)],
                    sem_out[k],
                )

            def compute(k, s):
                buf = bufs[k]
                w = _round16(chunk_lens[k])

                @plsc.parallel_loop(0, w, step=_LANES, unroll=8)
                def _(i):
                    o = pl.multiple_of(i, _LANES)
                    buf[pl.ds(o, _LANES)] = jnp.exp(buf[pl.ds(o, _LANES)] * s)

            live = [k for k in range(_NCHUNK) if chunk_lens[k] > 0]
            for k in live:
                in_copy(k).start()
            pltpu.sync_copy(scale, scale_v)
            s = scale_v[...]
            for k in live:
                in_copy(k).wait()
                compute(k, s)
                out_copy(k).start()
            for k in live:
                out_copy(k).wait()

        @pl.when(wid < _NUM_WORKERS - 1)
        def _():
            pipeline(wid * c_std, (cw,) * _NCHUNK)

        @pl.when(wid == _NUM_WORKERS - 1)
        def _():
            pipeline(last_start, last_lens)

    return run


def kernel(indices, values, rationality):
    del indices
    run = _build_sc_exp_map(values.shape[0])
    scale = jnp.full((_LANES,), -rationality, dtype=jnp.float32)
    return run(values, scale)
